# baseline (speedup 1.0000x reference)
import functools

import jax
import jax.numpy as jnp
from jax import lax
from jax.experimental import pallas as pl
from jax.experimental.pallas import tpu as pltpu
from jax.experimental.pallas import tpu_sc as plsc

_LANES = 16
_CHUNK = 64
_CP = 128


def _make_resample(B, H, W, C):
    N = H * W
    R = B * N
    info = plsc.get_sparse_core_info()
    NC, NS = info.num_cores, info.num_subcores
    NW = NC * NS
    assert R % NW == 0
    rows_per_tile = R // NW
    assert rows_per_tile % (2 * _CHUNK) == 0
    n_chunks = rows_per_tile // _CHUNK
    groups_per_chunk = _CHUNK // _LANES
    assert C <= _CP and _CP % _LANES == 0
    cvecs = C // _LANES
    assert cvecs * _LANES == C
    tiles_per_batch = N // rows_per_tile
    assert tiles_per_batch * rows_per_tile == N
    assert tiles_per_batch & (tiles_per_batch - 1) == 0
    tpb_shift = tiles_per_batch.bit_length() - 1
    assert W == 224 and H == 224

    mesh = plsc.VectorSubcoreMesh(core_axis_name="c", subcore_axis_name="s")

    nbuf_scratch = []
    for _ in range(2):
        nbuf_scratch += (
            [pltpu.VMEM((_CHUNK,), jnp.int32)] * 4
            + [pltpu.VMEM((_CHUNK,), jnp.float32)] * 2
            + [pltpu.VMEM((_CHUNK, _CP), jnp.float32)] * 5
            + [pltpu.SemaphoreType.DMA]
        )

    @functools.partial(
        pl.kernel,
        out_type=jax.ShapeDtypeStruct((R, _CP), jnp.float32),
        mesh=mesh,
        scratch_types=[
            pltpu.VMEM((rows_per_tile,), jnp.float32),
            pltpu.VMEM((rows_per_tile,), jnp.float32),
        ] + nbuf_scratch,
        compiler_params=pltpu.CompilerParams(needs_layout_passes=False),
    )
    def resample(oy_hbm, ox_hbm, x_hbm, out_hbm, oy_v, ox_v, *scr):
        wid = lax.axis_index("s") * NC + lax.axis_index("c")
        base = wid * rows_per_tile
        bidx = lax.shift_right_logical(wid, tpb_shift)
        bbase = bidx * N
        n0 = base - bbase

        bufs = [scr[p * 12:(p + 1) * 12] for p in range(2)]

        pltpu.sync_copy(oy_hbm.at[pl.ds(base, rows_per_tile)], oy_v)
        pltpu.sync_copy(ox_hbm.at[pl.ds(base, rows_per_tile)], ox_v)

        lane = jnp.arange(_LANES, dtype=jnp.int32)

        def fire(k, buf):
            ilt, irt, ilb, irb, u_v, v_v, lt_b, rt_b, lb_b, rb_b, _, sem = buf
            t0 = k * _CHUNK

            def idx_body(g, carry):
                loc = t0 + g * _LANES
                n = n0 + loc + lane
                gy = ((n >> 5) * 9363) >> 16
                gx = n - gy * W
                cy = jnp.clip(
                    gy.astype(jnp.float32) + oy_v[pl.ds(loc, _LANES)],
                    0.0, float(H) - 1.0)
                cx = jnp.clip(
                    gx.astype(jnp.float32) + ox_v[pl.ds(loc, _LANES)],
                    0.0, float(W) - 1.0)
                y0 = jnp.minimum(cy.astype(jnp.int32), H - 2)
                x0 = jnp.minimum(cx.astype(jnp.int32), W - 2)
                u = cy - y0.astype(jnp.float32)
                v = cx - x0.astype(jnp.float32)
                lt = bbase + y0 * W + x0
                s = pl.ds(g * _LANES, _LANES)
                ilt[s] = lt
                irt[s] = lt + 1
                ilb[s] = lt + W
                irb[s] = lt + (W + 1)
                u_v[s] = u
                v_v[s] = v
                return carry

            lax.fori_loop(0, groups_per_chunk, idx_body, 0)

            pltpu.async_copy(x_hbm.at[ilt], lt_b, sem)
            pltpu.async_copy(x_hbm.at[irt], rt_b, sem)
            pltpu.async_copy(x_hbm.at[ilb], lb_b, sem)
            pltpu.async_copy(x_hbm.at[irb], rb_b, sem)

        def drain_blend(k, buf):
            ilt, irt, ilb, irb, u_v, v_v, lt_b, rt_b, lb_b, rb_b, out_b, sem = buf
            pltpu.make_async_copy(x_hbm.at[ilt], lt_b, sem).wait()
            pltpu.make_async_copy(x_hbm.at[irt], rt_b, sem).wait()
            pltpu.make_async_copy(x_hbm.at[ilb], lb_b, sem).wait()
            pltpu.make_async_copy(x_hbm.at[irb], rb_b, sem).wait()

            def row(i):
                isplat = jnp.zeros((_LANES,), jnp.int32) + i
                u = plsc.load_gather(u_v, [isplat])
                v = plsc.load_gather(v_v, [isplat])
                for j in range(cvecs):
                    s = pl.ds(j * _LANES, _LANES)
                    lt = lt_b[i, s]
                    rt = rt_b[i, s]
                    lb = lb_b[i, s]
                    rb = rb_b[i, s]
                    top = lt + (rt - lt) * v
                    bot = lb + (rb - lb) * v
                    out_b[i, s] = top + (bot - top) * u

            def blend_body(h, carry):
                row(2 * h)
                row(2 * h + 1)
                return carry


            pltpu.sync_copy(out_b, out_hbm.at[pl.ds(base + k * _CHUNK, _CHUNK)])

        fire(0, bufs[0])

        def outer(j, carry):
            k0 = 2 * j
            fire(k0 + 1, bufs[1])
            drain_blend(k0, bufs[0])

            @pl.when(k0 + 2 < n_chunks)
            def _():
                fire(k0 + 2, bufs[0])

            drain_blend(k0 + 1, bufs[1])
            return carry

        lax.fori_loop(0, n_chunks // 2, outer, 0)

    return resample


_TR_BLOCK = 4


def _make_in_tr(C, W):
    def _in_tr(x_ref, o_ref):
        for t in range(_TR_BLOCK):
            tt = jnp.transpose(x_ref[t], (1, 0))
            o_ref[t] = jnp.concatenate(
                [tt, jnp.zeros((W, _CP - C), jnp.float32)], axis=1)
    return _in_tr


def _in_tr_tc(xt, C, W):
    S = xt.shape[0]
    return pl.pallas_call(
        _make_in_tr(C, W),
        grid=(S // _TR_BLOCK,),
        in_specs=[pl.BlockSpec((_TR_BLOCK, C, W), lambda i: (i, 0, 0))],
        out_specs=pl.BlockSpec((_TR_BLOCK, W, _CP), lambda i: (i, 0, 0)),
        out_shape=jax.ShapeDtypeStruct((S, W, _CP), jnp.float32),
    )(xt)


def _make_out_tr(C, W):
    def _out_tr(x_ref, o_ref):
        for t in range(_TR_BLOCK):
            o_ref[t] = jnp.transpose(x_ref[t, :, :C], (1, 0))
    return _out_tr


def _out_tr_tc(op, C, W):
    S = op.shape[0]
    return pl.pallas_call(
        _make_out_tr(C, W),
        grid=(S // _TR_BLOCK,),
        in_specs=[pl.BlockSpec((_TR_BLOCK, W, _CP), lambda i: (i, 0, 0))],
        out_specs=pl.BlockSpec((_TR_BLOCK, C, W), lambda i: (i, 0, 0)),
        out_shape=jax.ShapeDtypeStruct((S, C, W), jnp.float32),
    )(op)


def kernel(offsets, x):
    b, h, w, c = x.shape
    off2 = offsets.reshape(b * h * w, 2)
    oy = off2[:, 0]
    ox = off2[:, 1]
    xt = jnp.transpose(x, (0, 1, 3, 2)).reshape(b * h, c, w)
    xp = _in_tr_tc(xt, c, w).reshape(b * h * w, _CP)
    out = _make_resample(b, h, w, c)(oy, ox, xp)
    ot = _out_tr_tc(out.reshape(b * h, w, _CP), c, w)
    return jnp.transpose(ot.reshape(b, h, c, w), (0, 1, 3, 2))


# speedup vs baseline: 1.2365x; 1.2365x over previous
import functools

import jax
import jax.numpy as jnp
from jax import lax
from jax.experimental import pallas as pl
from jax.experimental.pallas import tpu as pltpu
from jax.experimental.pallas import tpu_sc as plsc

_LANES = 16
_CHUNK = 64
_CP = 128


def _make_resample(B, H, W, C):
    N = H * W
    R = B * N
    info = plsc.get_sparse_core_info()
    NC, NS = info.num_cores, info.num_subcores
    NW = NC * NS
    assert R % NW == 0
    rows_per_tile = R // NW
    assert rows_per_tile % (2 * _CHUNK) == 0
    n_chunks = rows_per_tile // _CHUNK
    groups_per_chunk = _CHUNK // _LANES
    assert C <= _CP and _CP % _LANES == 0
    cvecs = C // _LANES
    assert cvecs * _LANES == C
    tiles_per_batch = N // rows_per_tile
    assert tiles_per_batch * rows_per_tile == N
    assert tiles_per_batch & (tiles_per_batch - 1) == 0
    tpb_shift = tiles_per_batch.bit_length() - 1
    assert W == 224 and H == 224

    mesh = plsc.VectorSubcoreMesh(core_axis_name="c", subcore_axis_name="s")

    nbuf_scratch = []
    for _ in range(2):
        nbuf_scratch += (
            [pltpu.VMEM((_CHUNK,), jnp.int32)] * 4
            + [pltpu.VMEM((_CHUNK,), jnp.float32)] * 2
            + [pltpu.VMEM((_CHUNK, _CP), jnp.float32)] * 5
            + [pltpu.SemaphoreType.DMA] * 2
        )

    @functools.partial(
        pl.kernel,
        out_type=jax.ShapeDtypeStruct((R, _CP), jnp.float32),
        mesh=mesh,
        scratch_types=[
            pltpu.VMEM((rows_per_tile,), jnp.float32),
            pltpu.VMEM((rows_per_tile,), jnp.float32),
        ] + nbuf_scratch,
        compiler_params=pltpu.CompilerParams(needs_layout_passes=False),
    )
    def resample(oy_hbm, ox_hbm, x_hbm, out_hbm, oy_v, ox_v, *scr):
        wid = lax.axis_index("s") * NC + lax.axis_index("c")
        base = wid * rows_per_tile
        bidx = lax.shift_right_logical(wid, tpb_shift)
        bbase = bidx * N
        n0 = base - bbase

        bufs = [scr[p * 13:(p + 1) * 13] for p in range(2)]

        pltpu.sync_copy(oy_hbm.at[pl.ds(base, rows_per_tile)], oy_v)
        pltpu.sync_copy(ox_hbm.at[pl.ds(base, rows_per_tile)], ox_v)

        lane = jnp.arange(_LANES, dtype=jnp.int32)

        def fire(k, buf):
            ilt, irt, ilb, irb, u_v, v_v, lt_b, rt_b, lb_b, rb_b, _, sem, _2 = buf
            t0 = k * _CHUNK

            def idx_body(g, carry):
                loc = t0 + g * _LANES
                n = n0 + loc + lane
                gy = ((n >> 5) * 9363) >> 16
                gx = n - gy * W
                cy = jnp.clip(
                    gy.astype(jnp.float32) + oy_v[pl.ds(loc, _LANES)],
                    0.0, float(H) - 1.0)
                cx = jnp.clip(
                    gx.astype(jnp.float32) + ox_v[pl.ds(loc, _LANES)],
                    0.0, float(W) - 1.0)
                y0 = jnp.minimum(cy.astype(jnp.int32), H - 2)
                x0 = jnp.minimum(cx.astype(jnp.int32), W - 2)
                u = cy - y0.astype(jnp.float32)
                v = cx - x0.astype(jnp.float32)
                lt = bbase + y0 * W + x0
                s = pl.ds(g * _LANES, _LANES)
                ilt[s] = lt
                irt[s] = lt + 1
                ilb[s] = lt + W
                irb[s] = lt + (W + 1)
                u_v[s] = u
                v_v[s] = v
                return carry

            lax.fori_loop(0, groups_per_chunk, idx_body, 0)

            pltpu.async_copy(x_hbm.at[ilt], lt_b, sem)
            pltpu.async_copy(x_hbm.at[irt], rt_b, sem)
            pltpu.async_copy(x_hbm.at[ilb], lb_b, sem)
            pltpu.async_copy(x_hbm.at[irb], rb_b, sem)

        def drain_blend(k, buf):
            (ilt, irt, ilb, irb, u_v, v_v, lt_b, rt_b, lb_b, rb_b, out_b, sem,
             sem_o) = buf
            pltpu.make_async_copy(x_hbm.at[ilt], lt_b, sem).wait()
            pltpu.make_async_copy(x_hbm.at[irt], rt_b, sem).wait()
            pltpu.make_async_copy(x_hbm.at[ilb], lb_b, sem).wait()
            pltpu.make_async_copy(x_hbm.at[irb], rb_b, sem).wait()

            @pl.when(k >= 2)
            def _():
                pltpu.make_async_copy(
                    out_b,
                    out_hbm.at[pl.ds(base + (k - 2) * _CHUNK, _CHUNK)],
                    sem_o).wait()

            def row(i):
                isplat = jnp.zeros((_LANES,), jnp.int32) + i
                u = plsc.load_gather(u_v, [isplat])
                v = plsc.load_gather(v_v, [isplat])
                for j in range(cvecs):
                    s = pl.ds(j * _LANES, _LANES)
                    lt = lt_b[i, s]
                    rt = rt_b[i, s]
                    lb = lb_b[i, s]
                    rb = rb_b[i, s]
                    top = lt + (rt - lt) * v
                    bot = lb + (rb - lb) * v
                    out_b[i, s] = top + (bot - top) * u

            def blend_body(h, carry):
                for q in range(4):
                    row(4 * h + q)
                return carry

            lax.fori_loop(0, _CHUNK // 4, blend_body, 0)

            pltpu.async_copy(
                out_b, out_hbm.at[pl.ds(base + k * _CHUNK, _CHUNK)], sem_o)

        fire(0, bufs[0])

        def outer(j, carry):
            k0 = 2 * j
            fire(k0 + 1, bufs[1])
            drain_blend(k0, bufs[0])

            @pl.when(k0 + 2 < n_chunks)
            def _():
                fire(k0 + 2, bufs[0])

            drain_blend(k0 + 1, bufs[1])
            return carry

        lax.fori_loop(0, n_chunks // 2, outer, 0)

        for p, last_k in ((0, n_chunks - 2), (1, n_chunks - 1)):
            pltpu.make_async_copy(
                bufs[p][10],
                out_hbm.at[pl.ds(base + last_k * _CHUNK, _CHUNK)],
                bufs[p][12]).wait()

    return resample


_TR_BLOCK = 8


def _make_in_tr(C, W):
    def _in_tr(x_ref, o_ref):
        for t in range(_TR_BLOCK):
            tt = jnp.transpose(x_ref[t], (1, 0))
            o_ref[t] = jnp.concatenate(
                [tt, jnp.zeros((W, _CP - C), jnp.float32)], axis=1)
    return _in_tr


def _in_tr_tc(xt, C, W):
    S = xt.shape[0]
    return pl.pallas_call(
        _make_in_tr(C, W),
        grid=(S // _TR_BLOCK,),
        in_specs=[pl.BlockSpec((_TR_BLOCK, C, W), lambda i: (i, 0, 0))],
        out_specs=pl.BlockSpec((_TR_BLOCK, W, _CP), lambda i: (i, 0, 0)),
        out_shape=jax.ShapeDtypeStruct((S, W, _CP), jnp.float32),
    )(xt)


def _make_out_tr(C, W):
    def _out_tr(x_ref, o_ref):
        for t in range(_TR_BLOCK):
            o_ref[t] = jnp.transpose(x_ref[t, :, :C], (1, 0))
    return _out_tr


def _out_tr_tc(op, C, W):
    S = op.shape[0]
    return pl.pallas_call(
        _make_out_tr(C, W),
        grid=(S // _TR_BLOCK,),
        in_specs=[pl.BlockSpec((_TR_BLOCK, W, _CP), lambda i: (i, 0, 0))],
        out_specs=pl.BlockSpec((_TR_BLOCK, C, W), lambda i: (i, 0, 0)),
        out_shape=jax.ShapeDtypeStruct((S, C, W), jnp.float32),
    )(op)


def kernel(offsets, x):
    b, h, w, c = x.shape
    off2 = offsets.reshape(b * h * w, 2)
    oy = off2[:, 0]
    ox = off2[:, 1]
    xt = jnp.transpose(x, (0, 1, 3, 2)).reshape(b * h, c, w)
    xp = _in_tr_tc(xt, c, w).reshape(b * h * w, _CP)
    out = _make_resample(b, h, w, c)(oy, ox, xp)
    ot = _out_tr_tc(out.reshape(b * h, w, _CP), c, w)
    return jnp.transpose(ot.reshape(b, h, c, w), (0, 1, 3, 2))


# speedup vs baseline: 1.4394x; 1.1641x over previous
import functools

import jax
import jax.numpy as jnp
from jax import lax
from jax.experimental import pallas as pl
from jax.experimental.pallas import tpu as pltpu
from jax.experimental.pallas import tpu_sc as plsc

_LANES = 16
_CHUNK = 64
_CP = 128


def _make_resample(B, H, W, C):
    N = H * W
    R = B * N
    info = plsc.get_sparse_core_info()
    NC, NS = info.num_cores, info.num_subcores
    NW = NC * NS
    assert R % NW == 0
    rows_per_tile = R // NW
    assert rows_per_tile % (2 * _CHUNK) == 0
    n_chunks = rows_per_tile // _CHUNK
    groups_per_chunk = _CHUNK // _LANES
    assert C <= _CP and _CP % _LANES == 0
    cvecs = C // _LANES
    assert cvecs * _LANES == C
    tiles_per_batch = N // rows_per_tile
    assert tiles_per_batch * rows_per_tile == N
    assert tiles_per_batch & (tiles_per_batch - 1) == 0
    tpb_shift = tiles_per_batch.bit_length() - 1
    assert W == 224 and H == 224

    mesh = plsc.VectorSubcoreMesh(core_axis_name="c", subcore_axis_name="s")

    nbuf_scratch = []
    for _ in range(2):
        nbuf_scratch += (
            [pltpu.VMEM((_CHUNK,), jnp.int32)] * 4
            + [pltpu.VMEM((_CHUNK,), jnp.float32)] * 2
            + [pltpu.VMEM((_CHUNK, _CP), jnp.float32)] * 5
            + [pltpu.SemaphoreType.DMA] * 2
        )

    @functools.partial(
        pl.kernel,
        out_type=jax.ShapeDtypeStruct((R, _CP), jnp.float32),
        mesh=mesh,
        scratch_types=[
            pltpu.VMEM((rows_per_tile,), jnp.float32),
            pltpu.VMEM((rows_per_tile,), jnp.float32),
        ] + nbuf_scratch,
        compiler_params=pltpu.CompilerParams(needs_layout_passes=False),
    )
    def resample(oy_hbm, ox_hbm, x_hbm, out_hbm, oy_v, ox_v, *scr):
        wid = lax.axis_index("s") * NC + lax.axis_index("c")
        base = wid * rows_per_tile
        bidx = lax.shift_right_logical(wid, tpb_shift)
        bbase = bidx * N
        n0 = base - bbase

        bufs = [scr[p * 13:(p + 1) * 13] for p in range(2)]

        pltpu.sync_copy(oy_hbm.at[pl.ds(base, rows_per_tile)], oy_v)
        pltpu.sync_copy(ox_hbm.at[pl.ds(base, rows_per_tile)], ox_v)

        lane = jnp.arange(_LANES, dtype=jnp.int32)

        def fire(k, buf):
            ilt, irt, ilb, irb, u_v, v_v, lt_b, rt_b, lb_b, rb_b, _, sem, _2 = buf
            t0 = k * _CHUNK

            def idx_body(g, carry):
                loc = t0 + g * _LANES
                n = n0 + loc + lane
                gy = ((n >> 5) * 9363) >> 16
                gx = n - gy * W
                cy = jnp.clip(
                    gy.astype(jnp.float32) + oy_v[pl.ds(loc, _LANES)],
                    0.0, float(H) - 1.0)
                cx = jnp.clip(
                    gx.astype(jnp.float32) + ox_v[pl.ds(loc, _LANES)],
                    0.0, float(W) - 1.0)
                y0 = jnp.minimum(cy.astype(jnp.int32), H - 2)
                x0 = jnp.minimum(cx.astype(jnp.int32), W - 2)
                u = cy - y0.astype(jnp.float32)
                v = cx - x0.astype(jnp.float32)
                lt = bbase + y0 * W + x0
                s = pl.ds(g * _LANES, _LANES)
                ilt[s] = lt
                irt[s] = lt + 1
                ilb[s] = lt + W
                irb[s] = lt + (W + 1)
                u_v[s] = u
                v_v[s] = v
                return carry

            lax.fori_loop(0, groups_per_chunk, idx_body, 0)

            pltpu.async_copy(x_hbm.at[ilt], lt_b, sem)
            pltpu.async_copy(x_hbm.at[irt], rt_b, sem)
            pltpu.async_copy(x_hbm.at[ilb], lb_b, sem)
            pltpu.async_copy(x_hbm.at[irb], rb_b, sem)

        def drain_blend(k, buf):
            (ilt, irt, ilb, irb, u_v, v_v, lt_b, rt_b, lb_b, rb_b, out_b, sem,
             sem_o) = buf
            pltpu.make_async_copy(x_hbm.at[ilt], lt_b, sem).wait()
            pltpu.make_async_copy(x_hbm.at[irt], rt_b, sem).wait()
            pltpu.make_async_copy(x_hbm.at[ilb], lb_b, sem).wait()
            pltpu.make_async_copy(x_hbm.at[irb], rb_b, sem).wait()

            @pl.when(k >= 2)
            def _():
                pltpu.make_async_copy(
                    out_b,
                    out_hbm.at[pl.ds(base + (k - 2) * _CHUNK, _CHUNK)],
                    sem_o).wait()

            def row(i):
                isplat = jnp.zeros((_LANES,), jnp.int32) + i
                u = plsc.load_gather(u_v, [isplat])
                v = plsc.load_gather(v_v, [isplat])
                for j in range(cvecs):
                    s = pl.ds(j * _LANES, _LANES)
                    lt = lt_b[i, s]
                    rt = rt_b[i, s]
                    lb = lb_b[i, s]
                    rb = rb_b[i, s]
                    top = lt + (rt - lt) * v
                    bot = lb + (rb - lb) * v
                    out_b[i, s] = top + (bot - top) * u

            def blend_body(h, carry):
                for q in range(4):
                    row(4 * h + q)
                return carry

            lax.fori_loop(0, _CHUNK // 4, blend_body, 0)

            pltpu.async_copy(
                out_b, out_hbm.at[pl.ds(base + k * _CHUNK, _CHUNK)], sem_o)

        fire(0, bufs[0])

        def outer(j, carry):
            k0 = 2 * j
            fire(k0 + 1, bufs[1])
            drain_blend(k0, bufs[0])

            @pl.when(k0 + 2 < n_chunks)
            def _():
                fire(k0 + 2, bufs[0])

            drain_blend(k0 + 1, bufs[1])
            return carry

        lax.fori_loop(0, n_chunks // 2, outer, 0)

        for p, last_k in ((0, n_chunks - 2), (1, n_chunks - 1)):
            pltpu.make_async_copy(
                bufs[p][10],
                out_hbm.at[pl.ds(base + last_k * _CHUNK, _CHUNK)],
                bufs[p][12]).wait()

    return resample


_TR_BLOCK = 16


def _make_in_tr(C, W):
    def _in_tr(x_ref, o_ref):
        for t in range(_TR_BLOCK):
            tt = jnp.transpose(x_ref[t], (1, 0))
            o_ref[t] = jnp.concatenate(
                [tt, jnp.zeros((W, _CP - C), jnp.float32)], axis=1)
    return _in_tr


def _in_tr_tc(xt, C, W):
    S = xt.shape[0]
    return pl.pallas_call(
        _make_in_tr(C, W),
        grid=(S // _TR_BLOCK,),
        in_specs=[pl.BlockSpec((_TR_BLOCK, C, W), lambda i: (i, 0, 0))],
        out_specs=pl.BlockSpec((_TR_BLOCK, W, _CP), lambda i: (i, 0, 0)),
        out_shape=jax.ShapeDtypeStruct((S, W, _CP), jnp.float32),
    )(xt)


def _make_out_tr(C, W):
    def _out_tr(x_ref, o_ref):
        for t in range(_TR_BLOCK):
            o_ref[t] = jnp.transpose(x_ref[t, :, :C], (1, 0))
    return _out_tr


def _out_tr_tc(op, C, W):
    S = op.shape[0]
    return pl.pallas_call(
        _make_out_tr(C, W),
        grid=(S // _TR_BLOCK,),
        in_specs=[pl.BlockSpec((_TR_BLOCK, W, _CP), lambda i: (i, 0, 0))],
        out_specs=pl.BlockSpec((_TR_BLOCK, C, W), lambda i: (i, 0, 0)),
        out_shape=jax.ShapeDtypeStruct((S, C, W), jnp.float32),
    )(op)


def kernel(offsets, x):
    b, h, w, c = x.shape
    off2 = offsets.reshape(b * h * w, 2)
    oy = off2[:, 0]
    ox = off2[:, 1]
    xt = jnp.transpose(x, (0, 1, 3, 2)).reshape(b * h, c, w)
    xp = _in_tr_tc(xt, c, w).reshape(b * h * w, _CP)
    out = _make_resample(b, h, w, c)(oy, ox, xp)
    ot = _out_tr_tc(out.reshape(b * h, w, _CP), c, w)
    return jnp.transpose(ot.reshape(b, h, c, w), (0, 1, 3, 2))


# speedup vs baseline: 1.5642x; 1.0867x over previous
import functools

import jax
import jax.numpy as jnp
from jax import lax
from jax.experimental import pallas as pl
from jax.experimental.pallas import tpu as pltpu
from jax.experimental.pallas import tpu_sc as plsc

_LANES = 16
_CHUNK = 64
_CP = 128


def _make_resample(B, H, W, C):
    N = H * W
    R = B * N
    info = plsc.get_sparse_core_info()
    NC, NS = info.num_cores, info.num_subcores
    NW = NC * NS
    assert R % NW == 0
    rows_per_tile = R // NW
    assert rows_per_tile % (2 * _CHUNK) == 0
    n_chunks = rows_per_tile // _CHUNK
    groups_per_chunk = _CHUNK // _LANES
    assert C <= _CP and _CP % _LANES == 0
    cvecs = C // _LANES
    assert cvecs * _LANES == C
    tiles_per_batch = N // rows_per_tile
    assert tiles_per_batch * rows_per_tile == N
    assert tiles_per_batch & (tiles_per_batch - 1) == 0
    tpb_shift = tiles_per_batch.bit_length() - 1
    assert W == 224 and H == 224

    mesh = plsc.VectorSubcoreMesh(core_axis_name="c", subcore_axis_name="s")

    nbuf_scratch = []
    for _ in range(2):
        nbuf_scratch += (
            [pltpu.VMEM((_CHUNK,), jnp.int32)] * 4
            + [pltpu.VMEM((_CHUNK,), jnp.float32)] * 2
            + [pltpu.VMEM((_CHUNK, _CP), jnp.float32)] * 5
            + [pltpu.SemaphoreType.DMA] * 2
        )

    @functools.partial(
        pl.kernel,
        out_type=jax.ShapeDtypeStruct((R, _CP), jnp.float32),
        mesh=mesh,
        scratch_types=[
            pltpu.VMEM((rows_per_tile,), jnp.float32),
            pltpu.VMEM((rows_per_tile,), jnp.float32),
        ] + nbuf_scratch,
        compiler_params=pltpu.CompilerParams(needs_layout_passes=False),
    )
    def resample(oy_hbm, ox_hbm, x_hbm, out_hbm, oy_v, ox_v, *scr):
        wid = lax.axis_index("s") * NC + lax.axis_index("c")
        base = wid * rows_per_tile
        bidx = lax.shift_right_logical(wid, tpb_shift)
        bbase = bidx * N
        n0 = base - bbase

        bufs = [scr[p * 13:(p + 1) * 13] for p in range(2)]

        pltpu.sync_copy(oy_hbm.at[pl.ds(base, rows_per_tile)], oy_v)
        pltpu.sync_copy(ox_hbm.at[pl.ds(base, rows_per_tile)], ox_v)

        lane = jnp.arange(_LANES, dtype=jnp.int32)

        def fire(k, buf):
            ilt, irt, ilb, irb, u_v, v_v, lt_b, rt_b, lb_b, rb_b, _, sem, _2 = buf
            t0 = k * _CHUNK

            def idx_body(g, carry):
                loc = t0 + g * _LANES
                n = n0 + loc + lane
                gy = ((n >> 5) * 9363) >> 16
                gx = n - gy * W
                cy = jnp.clip(
                    gy.astype(jnp.float32) + oy_v[pl.ds(loc, _LANES)],
                    0.0, float(H) - 1.0)
                cx = jnp.clip(
                    gx.astype(jnp.float32) + ox_v[pl.ds(loc, _LANES)],
                    0.0, float(W) - 1.0)
                y0 = jnp.minimum(cy.astype(jnp.int32), H - 2)
                x0 = jnp.minimum(cx.astype(jnp.int32), W - 2)
                u = cy - y0.astype(jnp.float32)
                v = cx - x0.astype(jnp.float32)
                lt = bbase + y0 * W + x0
                s = pl.ds(g * _LANES, _LANES)
                ilt[s] = lt
                irt[s] = lt + 1
                ilb[s] = lt + W
                irb[s] = lt + (W + 1)
                u_v[s] = u
                v_v[s] = v
                return carry

            lax.fori_loop(0, groups_per_chunk, idx_body, 0)

            pltpu.async_copy(x_hbm.at[ilt], lt_b, sem)
            pltpu.async_copy(x_hbm.at[irt], rt_b, sem)
            pltpu.async_copy(x_hbm.at[ilb], lb_b, sem)
            pltpu.async_copy(x_hbm.at[irb], rb_b, sem)

        def drain_blend(k, buf):
            (ilt, irt, ilb, irb, u_v, v_v, lt_b, rt_b, lb_b, rb_b, out_b, sem,
             sem_o) = buf
            pltpu.make_async_copy(x_hbm.at[ilt], lt_b, sem).wait()
            pltpu.make_async_copy(x_hbm.at[irt], rt_b, sem).wait()
            pltpu.make_async_copy(x_hbm.at[ilb], lb_b, sem).wait()
            pltpu.make_async_copy(x_hbm.at[irb], rb_b, sem).wait()

            @pl.when(k >= 2)
            def _():
                pltpu.make_async_copy(
                    out_b,
                    out_hbm.at[pl.ds(base + (k - 2) * _CHUNK, _CHUNK)],
                    sem_o).wait()

            def row(i):
                isplat = jnp.zeros((_LANES,), jnp.int32) + i
                u = plsc.load_gather(u_v, [isplat])
                v = plsc.load_gather(v_v, [isplat])
                for j in range(cvecs):
                    s = pl.ds(j * _LANES, _LANES)
                    lt = lt_b[i, s]
                    rt = rt_b[i, s]
                    lb = lb_b[i, s]
                    rb = rb_b[i, s]
                    top = lt + (rt - lt) * v
                    bot = lb + (rb - lb) * v
                    out_b[i, s] = top + (bot - top) * u

            def blend_body(h, carry):
                for q in range(4):
                    row(4 * h + q)
                return carry

            lax.fori_loop(0, _CHUNK // 4, blend_body, 0)

            pltpu.async_copy(
                out_b, out_hbm.at[pl.ds(base + k * _CHUNK, _CHUNK)], sem_o)

        fire(0, bufs[0])

        def outer(j, carry):
            k0 = 2 * j
            fire(k0 + 1, bufs[1])
            drain_blend(k0, bufs[0])

            @pl.when(k0 + 2 < n_chunks)
            def _():
                fire(k0 + 2, bufs[0])

            drain_blend(k0 + 1, bufs[1])
            return carry

        lax.fori_loop(0, n_chunks // 2, outer, 0)

        for p, last_k in ((0, n_chunks - 2), (1, n_chunks - 1)):
            pltpu.make_async_copy(
                bufs[p][10],
                out_hbm.at[pl.ds(base + last_k * _CHUNK, _CHUNK)],
                bufs[p][12]).wait()

    return resample


_TR_BLOCK = 32


def _make_in_tr(C, W):
    def _in_tr(x_ref, o_ref):
        for t in range(_TR_BLOCK):
            tt = jnp.transpose(x_ref[t], (1, 0))
            o_ref[t] = jnp.concatenate(
                [tt, jnp.zeros((W, _CP - C), jnp.float32)], axis=1)
    return _in_tr


def _in_tr_tc(xt, C, W):
    S = xt.shape[0]
    return pl.pallas_call(
        _make_in_tr(C, W),
        grid=(S // _TR_BLOCK,),
        in_specs=[pl.BlockSpec((_TR_BLOCK, C, W), lambda i: (i, 0, 0))],
        out_specs=pl.BlockSpec((_TR_BLOCK, W, _CP), lambda i: (i, 0, 0)),
        out_shape=jax.ShapeDtypeStruct((S, W, _CP), jnp.float32),
    )(xt)


def _make_out_tr(C, W):
    def _out_tr(x_ref, o_ref):
        for t in range(_TR_BLOCK):
            o_ref[t] = jnp.transpose(x_ref[t, :, :C], (1, 0))
    return _out_tr


def _out_tr_tc(op, C, W):
    S = op.shape[0]
    return pl.pallas_call(
        _make_out_tr(C, W),
        grid=(S // _TR_BLOCK,),
        in_specs=[pl.BlockSpec((_TR_BLOCK, W, _CP), lambda i: (i, 0, 0))],
        out_specs=pl.BlockSpec((_TR_BLOCK, C, W), lambda i: (i, 0, 0)),
        out_shape=jax.ShapeDtypeStruct((S, C, W), jnp.float32),
    )(op)


def kernel(offsets, x):
    b, h, w, c = x.shape
    off2 = offsets.reshape(b * h * w, 2)
    oy = off2[:, 0]
    ox = off2[:, 1]
    xt = jnp.transpose(x, (0, 1, 3, 2)).reshape(b * h, c, w)
    xp = _in_tr_tc(xt, c, w).reshape(b * h * w, _CP)
    out = _make_resample(b, h, w, c)(oy, ox, xp)
    ot = _out_tr_tc(out.reshape(b * h, w, _CP), c, w)
    return jnp.transpose(ot.reshape(b, h, c, w), (0, 1, 3, 2))


# speedup vs baseline: 1.5990x; 1.0223x over previous
import functools

import jax
import jax.numpy as jnp
from jax import lax
from jax.experimental import pallas as pl
from jax.experimental.pallas import tpu as pltpu
from jax.experimental.pallas import tpu_sc as plsc

_LANES = 16
_CHUNK = 64
_CP = 128


def _make_resample(B, H, W, C):
    N = H * W
    R = B * N
    info = plsc.get_sparse_core_info()
    NC, NS = info.num_cores, info.num_subcores
    NW = NC * NS
    assert R % NW == 0
    rows_per_tile = R // NW
    assert rows_per_tile % (2 * _CHUNK) == 0
    n_chunks = rows_per_tile // _CHUNK
    groups_per_chunk = _CHUNK // _LANES
    assert C <= _CP and _CP % _LANES == 0
    cvecs = C // _LANES
    assert cvecs * _LANES == C
    tiles_per_batch = N // rows_per_tile
    assert tiles_per_batch * rows_per_tile == N
    assert tiles_per_batch & (tiles_per_batch - 1) == 0
    tpb_shift = tiles_per_batch.bit_length() - 1
    assert W == 224 and H == 224

    mesh = plsc.VectorSubcoreMesh(core_axis_name="c", subcore_axis_name="s")

    nbuf_scratch = []
    for _ in range(2):
        nbuf_scratch += (
            [pltpu.VMEM((_CHUNK,), jnp.int32)] * 4
            + [pltpu.VMEM((_CHUNK,), jnp.float32)] * 2
            + [pltpu.VMEM((_CHUNK, _CP), jnp.float32)] * 5
            + [pltpu.SemaphoreType.DMA] * 2
        )

    @functools.partial(
        pl.kernel,
        out_type=jax.ShapeDtypeStruct((R, _CP), jnp.float32),
        mesh=mesh,
        scratch_types=[
            pltpu.VMEM((rows_per_tile,), jnp.float32),
            pltpu.VMEM((rows_per_tile,), jnp.float32),
        ] + nbuf_scratch,
        compiler_params=pltpu.CompilerParams(needs_layout_passes=False),
    )
    def resample(oy_hbm, ox_hbm, x_hbm, out_hbm, oy_v, ox_v, *scr):
        wid = lax.axis_index("s") * NC + lax.axis_index("c")
        base = wid * rows_per_tile
        bidx = lax.shift_right_logical(wid, tpb_shift)
        bbase = bidx * N
        n0 = base - bbase

        bufs = [scr[p * 13:(p + 1) * 13] for p in range(2)]

        pltpu.sync_copy(oy_hbm.at[pl.ds(base, rows_per_tile)], oy_v)
        pltpu.sync_copy(ox_hbm.at[pl.ds(base, rows_per_tile)], ox_v)

        lane = jnp.arange(_LANES, dtype=jnp.int32)

        def fire(k, buf):
            ilt, irt, ilb, irb, u_v, v_v, lt_b, rt_b, lb_b, rb_b, _, sem, _2 = buf
            t0 = k * _CHUNK

            def idx_body(g, carry):
                loc = t0 + g * _LANES
                n = n0 + loc + lane
                gy = ((n >> 5) * 9363) >> 16
                gx = n - gy * W
                cy = jnp.clip(
                    gy.astype(jnp.float32) + oy_v[pl.ds(loc, _LANES)],
                    0.0, float(H) - 1.0)
                cx = jnp.clip(
                    gx.astype(jnp.float32) + ox_v[pl.ds(loc, _LANES)],
                    0.0, float(W) - 1.0)
                y0 = jnp.minimum(cy.astype(jnp.int32), H - 2)
                x0 = jnp.minimum(cx.astype(jnp.int32), W - 2)
                u = cy - y0.astype(jnp.float32)
                v = cx - x0.astype(jnp.float32)
                lt = bbase + y0 * W + x0
                s = pl.ds(g * _LANES, _LANES)
                ilt[s] = lt
                irt[s] = lt + 1
                ilb[s] = lt + W
                irb[s] = lt + (W + 1)
                u_v[s] = u
                v_v[s] = v
                return carry

            lax.fori_loop(0, groups_per_chunk, idx_body, 0)

            pltpu.async_copy(x_hbm.at[ilt], lt_b, sem)
            pltpu.async_copy(x_hbm.at[irt], rt_b, sem)
            pltpu.async_copy(x_hbm.at[ilb], lb_b, sem)
            pltpu.async_copy(x_hbm.at[irb], rb_b, sem)

        def drain_blend(k, buf):
            (ilt, irt, ilb, irb, u_v, v_v, lt_b, rt_b, lb_b, rb_b, out_b, sem,
             sem_o) = buf
            pltpu.make_async_copy(x_hbm.at[ilt], lt_b, sem).wait()
            pltpu.make_async_copy(x_hbm.at[irt], rt_b, sem).wait()
            pltpu.make_async_copy(x_hbm.at[ilb], lb_b, sem).wait()
            pltpu.make_async_copy(x_hbm.at[irb], rb_b, sem).wait()

            @pl.when(k >= 2)
            def _():
                pltpu.make_async_copy(
                    out_b,
                    out_hbm.at[pl.ds(base + (k - 2) * _CHUNK, _CHUNK)],
                    sem_o).wait()

            def row(i):
                isplat = jnp.zeros((_LANES,), jnp.int32) + i
                u = plsc.load_gather(u_v, [isplat])
                v = plsc.load_gather(v_v, [isplat])
                for j in range(cvecs):
                    s = pl.ds(j * _LANES, _LANES)
                    lt = lt_b[i, s]
                    rt = rt_b[i, s]
                    lb = lb_b[i, s]
                    rb = rb_b[i, s]
                    top = lt + (rt - lt) * v
                    bot = lb + (rb - lb) * v
                    out_b[i, s] = top + (bot - top) * u

            def blend_body(h, carry):
                for q in range(4):
                    row(4 * h + q)
                return carry

            lax.fori_loop(0, _CHUNK // 4, blend_body, 0)

            pltpu.async_copy(
                out_b, out_hbm.at[pl.ds(base + k * _CHUNK, _CHUNK)], sem_o)

        fire(0, bufs[0])

        def outer(j, carry):
            k0 = 2 * j
            fire(k0 + 1, bufs[1])
            drain_blend(k0, bufs[0])

            @pl.when(k0 + 2 < n_chunks)
            def _():
                fire(k0 + 2, bufs[0])

            drain_blend(k0 + 1, bufs[1])
            return carry

        lax.fori_loop(0, n_chunks // 2, outer, 0)

        for p, last_k in ((0, n_chunks - 2), (1, n_chunks - 1)):
            pltpu.make_async_copy(
                bufs[p][10],
                out_hbm.at[pl.ds(base + last_k * _CHUNK, _CHUNK)],
                bufs[p][12]).wait()

    return resample


_TR_BLOCK = 64


def _make_in_tr(C, W):
    def _in_tr(x_ref, o_ref):
        for t in range(_TR_BLOCK):
            tt = jnp.transpose(x_ref[t], (1, 0))
            o_ref[t] = jnp.concatenate(
                [tt, jnp.zeros((W, _CP - C), jnp.float32)], axis=1)
    return _in_tr


def _in_tr_tc(xt, C, W):
    S = xt.shape[0]
    return pl.pallas_call(
        _make_in_tr(C, W),
        grid=(S // _TR_BLOCK,),
        in_specs=[pl.BlockSpec((_TR_BLOCK, C, W), lambda i: (i, 0, 0))],
        out_specs=pl.BlockSpec((_TR_BLOCK, W, _CP), lambda i: (i, 0, 0)),
        out_shape=jax.ShapeDtypeStruct((S, W, _CP), jnp.float32),
    )(xt)


def _make_out_tr(C, W):
    def _out_tr(x_ref, o_ref):
        for t in range(_TR_BLOCK):
            o_ref[t] = jnp.transpose(x_ref[t, :, :C], (1, 0))
    return _out_tr


def _out_tr_tc(op, C, W):
    S = op.shape[0]
    return pl.pallas_call(
        _make_out_tr(C, W),
        grid=(S // _TR_BLOCK,),
        in_specs=[pl.BlockSpec((_TR_BLOCK, W, _CP), lambda i: (i, 0, 0))],
        out_specs=pl.BlockSpec((_TR_BLOCK, C, W), lambda i: (i, 0, 0)),
        out_shape=jax.ShapeDtypeStruct((S, C, W), jnp.float32),
    )(op)


def kernel(offsets, x):
    b, h, w, c = x.shape
    off2 = offsets.reshape(b * h * w, 2)
    oy = off2[:, 0]
    ox = off2[:, 1]
    xt = jnp.transpose(x, (0, 1, 3, 2)).reshape(b * h, c, w)
    xp = _in_tr_tc(xt, c, w).reshape(b * h * w, _CP)
    out = _make_resample(b, h, w, c)(oy, ox, xp)
    ot = _out_tr_tc(out.reshape(b * h, w, _CP), c, w)
    return jnp.transpose(ot.reshape(b, h, c, w), (0, 1, 3, 2))
